# baseline (speedup 1.0000x reference)
import functools
import math

import jax
from jax import lax
import jax.numpy as jnp
from jax.experimental import pallas as pl
from jax.experimental.pallas import tpu as pltpu
from jax.experimental.pallas import tpu_sc as plsc

_S = 64.0
_RB = 8

_NC, _NS = 2, 16
_NW = _NC * _NS
_CB = 1408
_NCH = 71
_NBUF = 4
_TAIL = 32


def _acos(x):
    p = jnp.float32(-0.0012624911)
    p = p * x + jnp.float32(0.0066700901)
    p = p * x - jnp.float32(0.0170881256)
    p = p * x + jnp.float32(0.0308918810)
    p = p * x - jnp.float32(0.0501743046)
    p = p * x + jnp.float32(0.0889789874)
    p = p * x - jnp.float32(0.2145988016)
    p = p * x + jnp.float32(1.5707963050)
    return jnp.sqrt(1.0 - x) * p


def _make_sc_scale(b, n):
    spt = (b // 8) // _NW
    total = _NCH * _NBUF
    mesh = plsc.VectorSubcoreMesh(core_axis_name="c", subcore_axis_name="s")

    def body(x_hbm, out_hbm, b0, b1, b2, b3, gsem, ssem):
        bufs = [b0, b1, b2, b3]
        wid = lax.axis_index("s") * _NC + lax.axis_index("c")
        base_stripe = wid * spt

        def addr(t):
            stripe_local = lax.bitwise_and(t, _NBUF - 1)
            k = lax.shift_right_logical(t, 2)
            return (base_stripe + stripe_local) * 8, k * _CB

        def gather(t, bi):
            row0, c0 = addr(t)
            return pltpu.make_async_copy(
                x_hbm.at[pl.ds(row0, 8), pl.ds(c0, _CB)], bufs[bi], gsem)

        def scatter(t, bi):
            row0, c0 = addr(t)
            return pltpu.make_async_copy(
                bufs[bi], out_hbm.at[pl.ds(row0, 8), pl.ds(c0, _CB)], ssem)

        gather(0, 0).start()
        gather(1, 1).start()

        def outer(g, carry):
            for bi in range(_NBUF):
                t = g * _NBUF + bi

                @pl.when(t >= 2)
                def _():
                    scatter(t - 2, (bi + 2) % _NBUF).wait()

                @pl.when(t + 2 < total)
                def _():
                    gather(t + 2, (bi + 2) % _NBUF).start()

                gather(t, bi).wait()

                for r in range(8):
                    def mul_body(i, c2, buf=bufs[bi], r=r):
                        sl = pl.ds(lax.mul(i, 16), 16)
                        buf[r, sl] = buf[r, sl] * _S
                        return c2

                    lax.fori_loop(0, _CB // 16, mul_body, 0, unroll=8)
                scatter(t, bi).start()
            return carry

        lax.fori_loop(0, total // _NBUF, outer, 0)
        scatter(total - 2, (total - 2) % _NBUF).wait()
        scatter(total - 1, (total - 1) % _NBUF).wait()

    return pl.kernel(
        body,
        out_type=jax.ShapeDtypeStruct((b, n), jnp.float32),
        mesh=mesh,
        scratch_types=[pltpu.VMEM((8, _CB), jnp.float32)] * _NBUF
        + [pltpu.SemaphoreType.DMA, pltpu.SemaphoreType.DMA],
        compiler_params=pltpu.CompilerParams(use_tc_tiling_on_sc=True),
    )


def _sum_body(lab_ref, x_ref, tgt_ref, win_ref, sum_ref):
    i = pl.program_id(0)
    x = x_ref[...]

    @pl.when(i == 0)
    def _():
        sum_ref[0, 0] = 0.0

    sum_ref[0, 0] += jnp.sum(x)

    tiles = []
    lanes = []
    for r in range(_RB):
        col = lab_ref[i * _RB + r]
        col_tile = pl.multiple_of((col // 128) * 128, 128)
        tiles.append(x_ref[pl.ds(r, 1), pl.ds(col_tile, 128)])
        lanes.append(col - col_tile)
    win = jnp.concatenate(tiles, axis=0)
    win_ref[...] = win * _S
    lane = jnp.concatenate([jnp.full((1, 1), l, jnp.int32) for l in lanes], axis=0)
    lane_iota = jax.lax.broadcasted_iota(jnp.int32, (_RB, 128), 1)
    picked = jnp.where(lane_iota == lane, win, 0.0)
    tgt_ref[0, :, :] = jnp.sum(picked, axis=1, keepdims=True)


def _fixup_body(out_in, tgt_ref, win_ref, lab2_ref, tail_ref, tot_ref, lab_ref,
                out_hbm, blend_ref, tail_out_ref, sem):
    del out_in
    b, n = out_hbm.shape

    tail_out_ref[...] = tail_ref[...] * _S
    tail_copy = pltpu.make_async_copy(
        tail_out_ref, out_hbm.at[:, pl.ds(n - _TAIL, _TAIL)], sem)
    tail_copy.start()
    t = tgt_ref[...]
    pos_sum = jnp.sum(t)
    total = tot_ref[0, 0]
    pos_mean = pos_sum / b
    neg_mean = (total - pos_sum) / (b * (n - 1))
    avg_p_theta = _acos(pos_mean)
    c = jnp.float32(math.log(n - 1) / _S)
    d_m = jnp.log(_acos(neg_mean + c) / math.pi) / jnp.log(avg_p_theta / math.pi)
    theta = _acos(t)
    ratio = theta * jnp.float32(1.0 / math.pi)
    warped = jnp.exp(d_m * jnp.log(ratio)) * jnp.float32(math.pi)
    final = jnp.cos(warped) * _S

    lane = jax.lax.rem(lab2_ref[...], jnp.int32(128))
    lane_iota = jax.lax.broadcasted_iota(jnp.int32, (b, 128), 1)
    blend_ref[...] = jnp.where(lane_iota == lane, final, win_ref[...])

    tail_copy.wait()

    def _copy(i):
        col = lab_ref[i]
        col_tile = pl.multiple_of((col // 128) * 128, 128)
        return pltpu.make_async_copy(
            blend_ref.at[pl.ds(i, 1), :],
            out_hbm.at[pl.ds(i, 1), pl.ds(col_tile, 128)],
            sem,
        )

    def _start(i, _):
        _copy(i).start()
        return 0

    def _wait(i, _):
        _copy(i).wait()
        return 0

    jax.lax.fori_loop(0, b, _start, 0)
    jax.lax.fori_loop(0, b, _wait, 0)


@jax.jit
def kernel(logits, labels):
    b, n = logits.shape
    nb = b // _RB

    out0 = _make_sc_scale(b, n)(logits)

    tgt3, wins, total = pl.pallas_call(
        _sum_body,
        grid=(nb,),
        in_specs=[
            pl.BlockSpec(memory_space=pltpu.SMEM),
            pl.BlockSpec((_RB, n), lambda i: (i, 0)),
        ],
        out_specs=[
            pl.BlockSpec((1, _RB, 1), lambda i: (i, 0, 0)),
            pl.BlockSpec((_RB, 128), lambda i: (i, 0)),
            pl.BlockSpec(memory_space=pltpu.SMEM),
        ],
        out_shape=[
            jax.ShapeDtypeStruct((nb, _RB, 1), jnp.float32),
            jax.ShapeDtypeStruct((b, 128), jnp.float32),
            jax.ShapeDtypeStruct((1, 1), jnp.float32),
        ],
    )(labels, logits)

    tgt = tgt3.reshape(b, 1)
    lab2 = labels.reshape(b, 1)

    out = pl.pallas_call(
        _fixup_body,
        in_specs=[
            pl.BlockSpec(memory_space=pl.ANY),
            pl.BlockSpec(memory_space=pltpu.VMEM),
            pl.BlockSpec(memory_space=pltpu.VMEM),
            pl.BlockSpec(memory_space=pltpu.VMEM),
            pl.BlockSpec(memory_space=pltpu.VMEM),
            pl.BlockSpec(memory_space=pltpu.SMEM),
            pl.BlockSpec(memory_space=pltpu.SMEM),
        ],
        out_specs=pl.BlockSpec(memory_space=pl.ANY),
        out_shape=jax.ShapeDtypeStruct((b, n), jnp.float32),
        input_output_aliases={0: 0},
        scratch_shapes=[
            pltpu.VMEM((b, 128), jnp.float32),
            pltpu.VMEM((b, _TAIL), jnp.float32),
            pltpu.SemaphoreType.DMA,
        ],
    )(out0, tgt, wins, lab2, logits[:, n - _TAIL:], total, labels)
    return out


# speedup vs baseline: 1.1631x; 1.1631x over previous
import functools
import math

import jax
from jax import lax
import jax.numpy as jnp
from jax.experimental import pallas as pl
from jax.experimental.pallas import tpu as pltpu

_S = 64.0
_RB = 8
_NBUF = 4


def _acos(x):
    p = jnp.float32(-0.0012624911)
    p = p * x + jnp.float32(0.0066700901)
    p = p * x - jnp.float32(0.0170881256)
    p = p * x + jnp.float32(0.0308918810)
    p = p * x - jnp.float32(0.0501743046)
    p = p * x + jnp.float32(0.0889789874)
    p = p * x - jnp.float32(0.2145988016)
    p = p * x + jnp.float32(1.5707963050)
    return jnp.sqrt(1.0 - x) * p


def _main_body(lab_ref, x_hbm, out_hbm, tgt_ref, win_ref, sum_ref,
               b0, b1, b2, b3, acc_ref, gsem, ssem):
    bufs = [b0, b1, b2, b3]
    b, n = x_hbm.shape
    nstripes = b // _RB

    def gather(t, bi):
        row0 = pl.multiple_of(t * _RB, _RB)
        return pltpu.make_async_copy(
            x_hbm.at[pl.ds(row0, _RB), :], bufs[bi], gsem)

    def scatter(t, bi):
        row0 = pl.multiple_of(t * _RB, _RB)
        return pltpu.make_async_copy(
            bufs[bi], out_hbm.at[pl.ds(row0, _RB), :], ssem)

    acc_ref[0] = 0.0
    gather(0, 0).start()
    gather(1, 1).start()

    def outer(g, carry):
        for bi in range(_NBUF):
            t = g * _NBUF + bi

            @pl.when(t >= 2)
            def _():
                scatter(t - 2, (bi + 2) % _NBUF).wait()

            @pl.when(t + 2 < nstripes)
            def _():
                gather(t + 2, (bi + 2) % _NBUF).start()

            gather(t, bi).wait()

            buf = bufs[bi]
            x = buf[...]
            acc_ref[0] += jnp.sum(x)

            tiles = []
            lanes = []
            for r in range(_RB):
                col = lab_ref[t * _RB + r]
                col_tile = pl.multiple_of((col // 128) * 128, 128)
                tiles.append(buf[pl.ds(r, 1), pl.ds(col_tile, 128)])
                lanes.append(col - col_tile)
            win = jnp.concatenate(tiles, axis=0)
            row0 = pl.multiple_of(t * _RB, _RB)
            win_ref[pl.ds(row0, _RB), :] = win * _S
            lane = jnp.concatenate(
                [jnp.full((1, 1), l, jnp.int32) for l in lanes], axis=0)
            lane_iota = jax.lax.broadcasted_iota(jnp.int32, (_RB, 128), 1)
            picked = jnp.where(lane_iota == lane, win, 0.0)
            tgt_ref[pl.ds(row0, _RB), :] = jnp.sum(picked, axis=1, keepdims=True)

            buf[...] = x * _S
            scatter(t, bi).start()
        return carry

    lax.fori_loop(0, nstripes // _NBUF, outer, 0)
    sum_ref[0, 0] = acc_ref[0]
    scatter(nstripes - 2, (nstripes - 2) % _NBUF).wait()
    scatter(nstripes - 1, (nstripes - 1) % _NBUF).wait()


def _fixup_body(out_in, tgt_ref, win_ref, lab2_ref, tot_ref, lab_ref, out_hbm,
                blend_ref, sem):
    del out_in
    b, n = out_hbm.shape
    t = tgt_ref[...]
    pos_sum = jnp.sum(t)
    total = tot_ref[0, 0]
    pos_mean = pos_sum / b
    neg_mean = (total - pos_sum) / (b * (n - 1))
    avg_p_theta = _acos(pos_mean)
    c = jnp.float32(math.log(n - 1) / _S)
    d_m = jnp.log(_acos(neg_mean + c) / math.pi) / jnp.log(avg_p_theta / math.pi)
    theta = _acos(t)
    ratio = theta * jnp.float32(1.0 / math.pi)
    warped = jnp.exp(d_m * jnp.log(ratio)) * jnp.float32(math.pi)
    final = jnp.cos(warped) * _S

    lane = jax.lax.rem(lab2_ref[...], jnp.int32(128))
    lane_iota = jax.lax.broadcasted_iota(jnp.int32, (b, 128), 1)
    blend_ref[...] = jnp.where(lane_iota == lane, final, win_ref[...])

    def _copy(i):
        col = lab_ref[i]
        col_tile = pl.multiple_of((col // 128) * 128, 128)
        return pltpu.make_async_copy(
            blend_ref.at[pl.ds(i, 1), :],
            out_hbm.at[pl.ds(i, 1), pl.ds(col_tile, 128)],
            sem,
        )

    def _start(i, _):
        _copy(i).start()
        return 0

    def _wait(i, _):
        _copy(i).wait()
        return 0

    jax.lax.fori_loop(0, b, _start, 0)
    jax.lax.fori_loop(0, b, _wait, 0)


@jax.jit
def kernel(logits, labels):
    b, n = logits.shape

    out0, tgt, wins, total = pl.pallas_call(
        _main_body,
        in_specs=[
            pl.BlockSpec(memory_space=pltpu.SMEM),
            pl.BlockSpec(memory_space=pl.ANY),
        ],
        out_specs=[
            pl.BlockSpec(memory_space=pl.ANY),
            pl.BlockSpec(memory_space=pltpu.VMEM),
            pl.BlockSpec(memory_space=pltpu.VMEM),
            pl.BlockSpec(memory_space=pltpu.SMEM),
        ],
        out_shape=[
            jax.ShapeDtypeStruct((b, n), jnp.float32),
            jax.ShapeDtypeStruct((b, 1), jnp.float32),
            jax.ShapeDtypeStruct((b, 128), jnp.float32),
            jax.ShapeDtypeStruct((1, 1), jnp.float32),
        ],
        scratch_shapes=[pltpu.VMEM((_RB, n), jnp.float32)] * _NBUF
        + [pltpu.SMEM((1,), jnp.float32),
           pltpu.SemaphoreType.DMA, pltpu.SemaphoreType.DMA],
    )(labels, logits)

    lab2 = labels.reshape(b, 1)

    out = pl.pallas_call(
        _fixup_body,
        in_specs=[
            pl.BlockSpec(memory_space=pl.ANY),
            pl.BlockSpec(memory_space=pltpu.VMEM),
            pl.BlockSpec(memory_space=pltpu.VMEM),
            pl.BlockSpec(memory_space=pltpu.VMEM),
            pl.BlockSpec(memory_space=pltpu.SMEM),
            pl.BlockSpec(memory_space=pltpu.SMEM),
        ],
        out_specs=pl.BlockSpec(memory_space=pl.ANY),
        out_shape=jax.ShapeDtypeStruct((b, n), jnp.float32),
        input_output_aliases={0: 0},
        scratch_shapes=[
            pltpu.VMEM((b, 128), jnp.float32),
            pltpu.SemaphoreType.DMA,
        ],
    )(out0, tgt, wins, lab2, total, labels)
    return out


# speedup vs baseline: 1.1669x; 1.0032x over previous
import functools
import math

import jax
from jax import lax
import jax.numpy as jnp
from jax.experimental import pallas as pl
from jax.experimental.pallas import tpu as pltpu

_S = 64.0
_RB = 8
_NBUF = 16
_LOOK = 8


def _acos(x):
    p = jnp.float32(-0.0012624911)
    p = p * x + jnp.float32(0.0066700901)
    p = p * x - jnp.float32(0.0170881256)
    p = p * x + jnp.float32(0.0308918810)
    p = p * x - jnp.float32(0.0501743046)
    p = p * x + jnp.float32(0.0889789874)
    p = p * x - jnp.float32(0.2145988016)
    p = p * x + jnp.float32(1.5707963050)
    return jnp.sqrt(1.0 - x) * p


def _main_body(lab_ref, x_hbm, out_hbm, tgt_ref, win_ref, sum_ref,
               *rest):
    bufs = list(rest[:_NBUF])
    acc_ref, gsem, ssem = rest[_NBUF:]
    b, n = x_hbm.shape
    nstripes = b // _RB

    def gather(t, bi):
        row0 = pl.multiple_of(t * _RB, _RB)
        return pltpu.make_async_copy(
            x_hbm.at[pl.ds(row0, _RB), :], bufs[bi], gsem)

    def scatter(t, bi):
        row0 = pl.multiple_of(t * _RB, _RB)
        return pltpu.make_async_copy(
            bufs[bi], out_hbm.at[pl.ds(row0, _RB), :], ssem)

    acc_ref[0] = 0.0
    for t0 in range(_LOOK):
        gather(t0, t0).start()

    def outer(g, carry):
        for bi in range(_NBUF):
            t = g * _NBUF + bi

            @pl.when(t >= _NBUF - _LOOK)
            def _():
                scatter(t - (_NBUF - _LOOK), (bi + _LOOK) % _NBUF).wait()

            @pl.when(t + _LOOK < nstripes)
            def _():
                gather(t + _LOOK, (bi + _LOOK) % _NBUF).start()

            gather(t, bi).wait()

            buf = bufs[bi]
            x = buf[...]
            acc_ref[0] += jnp.sum(x)

            tiles = []
            lanes = []
            for r in range(_RB):
                col = lab_ref[t * _RB + r]
                col_tile = pl.multiple_of((col // 128) * 128, 128)
                tiles.append(buf[pl.ds(r, 1), pl.ds(col_tile, 128)])
                lanes.append(col - col_tile)
            win = jnp.concatenate(tiles, axis=0)
            row0 = pl.multiple_of(t * _RB, _RB)
            win_ref[pl.ds(row0, _RB), :] = win * _S
            lane = jnp.concatenate(
                [jnp.full((1, 1), l, jnp.int32) for l in lanes], axis=0)
            lane_iota = jax.lax.broadcasted_iota(jnp.int32, (_RB, 128), 1)
            picked = jnp.where(lane_iota == lane, win, 0.0)
            tgt_ref[pl.ds(row0, _RB), :] = jnp.sum(picked, axis=1, keepdims=True)

            buf[...] = x * _S
            scatter(t, bi).start()
        return carry

    lax.fori_loop(0, nstripes // _NBUF, outer, 0)
    sum_ref[0, 0] = acc_ref[0]
    for td in range(nstripes - (_NBUF - _LOOK), nstripes):
        scatter(td, td % _NBUF).wait()


def _fixup_body(out_in, tgt_ref, win_ref, lab2_ref, tot_ref, lab_ref, out_hbm,
                blend_ref, sem):
    del out_in
    b, n = out_hbm.shape
    t = tgt_ref[...]
    pos_sum = jnp.sum(t)
    total = tot_ref[0, 0]
    pos_mean = pos_sum / b
    neg_mean = (total - pos_sum) / (b * (n - 1))
    avg_p_theta = _acos(pos_mean)
    c = jnp.float32(math.log(n - 1) / _S)
    d_m = jnp.log(_acos(neg_mean + c) / math.pi) / jnp.log(avg_p_theta / math.pi)
    theta = _acos(t)
    ratio = theta * jnp.float32(1.0 / math.pi)
    warped = jnp.exp(d_m * jnp.log(ratio)) * jnp.float32(math.pi)
    final = jnp.cos(warped) * _S

    lane = jax.lax.rem(lab2_ref[...], jnp.int32(128))
    lane_iota = jax.lax.broadcasted_iota(jnp.int32, (b, 128), 1)
    blend_ref[...] = jnp.where(lane_iota == lane, final, win_ref[...])

    def _copy(i):
        col = lab_ref[i]
        col_tile = pl.multiple_of((col // 128) * 128, 128)
        return pltpu.make_async_copy(
            blend_ref.at[pl.ds(i, 1), :],
            out_hbm.at[pl.ds(i, 1), pl.ds(col_tile, 128)],
            sem,
        )

    def _start(i, _):
        _copy(i).start()
        return 0

    def _wait(i, _):
        _copy(i).wait()
        return 0

    jax.lax.fori_loop(0, b, _start, 0)
    jax.lax.fori_loop(0, b, _wait, 0)


@jax.jit
def kernel(logits, labels):
    b, n = logits.shape

    out0, tgt, wins, total = pl.pallas_call(
        _main_body,
        in_specs=[
            pl.BlockSpec(memory_space=pltpu.SMEM),
            pl.BlockSpec(memory_space=pl.ANY),
        ],
        out_specs=[
            pl.BlockSpec(memory_space=pl.ANY),
            pl.BlockSpec(memory_space=pltpu.VMEM),
            pl.BlockSpec(memory_space=pltpu.VMEM),
            pl.BlockSpec(memory_space=pltpu.SMEM),
        ],
        out_shape=[
            jax.ShapeDtypeStruct((b, n), jnp.float32),
            jax.ShapeDtypeStruct((b, 1), jnp.float32),
            jax.ShapeDtypeStruct((b, 128), jnp.float32),
            jax.ShapeDtypeStruct((1, 1), jnp.float32),
        ],
        scratch_shapes=[pltpu.VMEM((_RB, n), jnp.float32)] * _NBUF
        + [pltpu.SMEM((1,), jnp.float32),
           pltpu.SemaphoreType.DMA, pltpu.SemaphoreType.DMA],
        compiler_params=pltpu.CompilerParams(
            vmem_limit_bytes=100 * 1024 * 1024),
    )(labels, logits)

    lab2 = labels.reshape(b, 1)

    out = pl.pallas_call(
        _fixup_body,
        in_specs=[
            pl.BlockSpec(memory_space=pl.ANY),
            pl.BlockSpec(memory_space=pltpu.VMEM),
            pl.BlockSpec(memory_space=pltpu.VMEM),
            pl.BlockSpec(memory_space=pltpu.VMEM),
            pl.BlockSpec(memory_space=pltpu.SMEM),
            pl.BlockSpec(memory_space=pltpu.SMEM),
        ],
        out_specs=pl.BlockSpec(memory_space=pl.ANY),
        out_shape=jax.ShapeDtypeStruct((b, n), jnp.float32),
        input_output_aliases={0: 0},
        scratch_shapes=[
            pltpu.VMEM((b, 128), jnp.float32),
            pltpu.SemaphoreType.DMA,
        ],
    )(out0, tgt, wins, lab2, total, labels)
    return out
